# baseline (speedup 1.0000x reference)
import functools

import jax
import jax.numpy as jnp
from jax import lax
from jax.experimental import pallas as pl
from jax.experimental.pallas import tpu as pltpu
from jax.experimental.pallas import tpu_sc as plsc

_NUM_CORES = 2
_NUM_SUBCORES = 16
_NUM_WORKERS = _NUM_CORES * _NUM_SUBCORES
_LANES = 16
_CHUNK = 128
_PACK_BLK = 1000


def _pack_halves_tc(data):
    num_rows, row_dim = data.shape
    half = num_rows // 2

    def body(lo_ref, hi_ref, out_ref):
        out_ref[:, :row_dim] = lo_ref[...]
        out_ref[:, row_dim:] = hi_ref[...]

    return pl.pallas_call(
        body,
        grid=(half // _PACK_BLK,),
        in_specs=[
            pl.BlockSpec((_PACK_BLK, row_dim), lambda i: (i, 0)),
            pl.BlockSpec((_PACK_BLK, row_dim), lambda i: (i + half // _PACK_BLK, 0)),
        ],
        out_specs=pl.BlockSpec((_PACK_BLK, 2 * row_dim), lambda i: (i, 0)),
        out_shape=jax.ShapeDtypeStruct((half, 2 * row_dim), data.dtype),
    )(data, data)


def kernel(data, indices):
    num_rows, row_dim = data.shape
    half = num_rows // 2
    batch = indices.shape[0]
    idx = indices.reshape(batch).astype(jnp.int32)
    packed = _pack_halves_tc(data)
    b_per_w = batch // _NUM_WORKERS
    n_chunks = b_per_w // _CHUNK

    mesh = plsc.VectorSubcoreMesh(core_axis_name="c", subcore_axis_name="s")

    @functools.partial(
        pl.kernel,
        mesh=mesh,
        out_type=jax.ShapeDtypeStruct((batch, row_dim), data.dtype),
        compiler_params=pltpu.CompilerParams(needs_layout_passes=False),
        scratch_types=[
            pltpu.VMEM((b_per_w,), jnp.int32),
            pltpu.VMEM((b_per_w,), jnp.int32),
        ]
        + [pltpu.VMEM((_CHUNK, 2 * row_dim), jnp.float32)
           for _ in range(n_chunks)]
        + [
            pltpu.VMEM((_CHUNK, row_dim), jnp.float32),
            pltpu.SemaphoreType.DMA((n_chunks,)),
        ],
    )
    def gather_rows_sc(packed_hbm, idx_hbm, out_hbm, idx_v, slot_v, *rest):
        rows_bufs = rest[:n_chunks]
        out_v = rest[n_chunks]
        sems = rest[n_chunks + 1]
        wid = lax.axis_index("s") * _NUM_CORES + lax.axis_index("c")
        base = wid * b_per_w
        pltpu.sync_copy(idx_hbm.at[pl.ds(base, b_per_w)], idx_v)

        @pl.loop(0, b_per_w, step=_LANES)
        def _(g):
            iv = idx_v[pl.ds(g, _LANES)]
            in_hi = (iv >= half).astype(jnp.int32)
            slot_v[pl.ds(g, _LANES)] = iv - in_hi * half

        copies = []
        for c in range(n_chunks):
            copies.append(pltpu.async_copy(
                packed_hbm.at[slot_v.at[pl.ds(c * _CHUNK, _CHUNK)]],
                rows_bufs[c],
                sems.at[c],
            ))

        row_iota = lax.iota(jnp.int32, _LANES)
        zero_v = jnp.zeros((_LANES,), jnp.int32)

        for c in range(n_chunks):
            copies[c].wait()
            rows_v = rows_bufs[c]

            @pl.loop(0, _CHUNK, step=_LANES)
            def _(g):
                iv = idx_v[pl.ds(c * _CHUNK + g, _LANES)]
                col0 = (iv >= half).astype(jnp.int32) * row_dim
                slots = row_iota + g
                for j in range(row_dim):
                    v = plsc.load_gather(rows_v, [slots, col0 + j])
                    plsc.store_scatter(out_v, [slots, zero_v + j], v)

            pltpu.sync_copy(
                out_v, out_hbm.at[pl.ds(base + c * _CHUNK, _CHUNK)]
            )

    return gather_rows_sc(packed, idx)


# speedup vs baseline: 2.3021x; 2.3021x over previous
import functools

import jax
import jax.numpy as jnp
from jax import lax
from jax.experimental import pallas as pl
from jax.experimental.pallas import tpu as pltpu
from jax.experimental.pallas import tpu_sc as plsc

_NUM_CORES = 2
_NUM_SUBCORES = 16
_NUM_WORKERS = _NUM_CORES * _NUM_SUBCORES
_LANES = 16


def kernel(data, indices):
    num_rows, row_dim = data.shape
    batch = indices.shape[0]
    idx = indices.reshape(batch).astype(jnp.int32)
    b_per_w = batch // _NUM_WORKERS

    mesh = plsc.VectorSubcoreMesh(core_axis_name="c", subcore_axis_name="s")

    @functools.partial(
        pl.kernel,
        mesh=mesh,
        out_type=jax.ShapeDtypeStruct((batch, row_dim), data.dtype),
        scratch_types=[
            pltpu.VMEM((b_per_w,), jnp.int32),
            pltpu.VMEM((b_per_w, row_dim), jnp.float32),
            pltpu.SemaphoreType.DMA,
        ],
    )
    def gather_rows_sc(table_hbm, idx_hbm, out_hbm, idx_v, rows_v, sem):
        wid = lax.axis_index("s") * _NUM_CORES + lax.axis_index("c")
        base = wid * b_per_w
        pltpu.sync_copy(idx_hbm.at[pl.ds(base, b_per_w)], idx_v)

        @pl.loop(0, b_per_w, step=_LANES)
        def _(g):
            vec = idx_v[pl.ds(g, _LANES)]
            for j in range(_LANES):
                pltpu.async_copy(
                    table_hbm.at[pl.ds(vec[j], 1)],
                    rows_v.at[pl.ds(g + j, 1)],
                    sem,
                )

        pltpu.make_async_copy(
            table_hbm.at[pl.ds(0, b_per_w)],
            rows_v,
            sem,
        ).wait()

        pltpu.sync_copy(rows_v, out_hbm.at[pl.ds(base, b_per_w)])

    return gather_rows_sc(data, idx)
